# Initial kernel scaffold; baseline (speedup 1.0000x reference)
import jax
import jax.numpy as jnp
from jax.experimental import pallas as pl


def kernel(input, edge_index, edge_weight, W, b):
    raise NotImplementedError("write your pallas kernel here")


# speedup vs baseline: 4.5881x; 4.5881x over previous
import functools

import jax
import jax.numpy as jnp
from jax import lax
from jax.experimental import pallas as pl
from jax.experimental.pallas import tpu as pltpu
from jax.experimental.pallas import tpu_sc as plsc

_NC = 2
_NS = 16
_L = 16
_CHUNK = 128


def _matmul(x, W):
    n, d_in = x.shape
    d_out = W.shape[1]
    blk = 1000

    def body(x_ref, w_ref, o_ref):
        o_ref[...] = jnp.dot(x_ref[...], w_ref[...],
                             preferred_element_type=jnp.float32)

    return pl.pallas_call(
        body,
        grid=(n // blk,),
        in_specs=[pl.BlockSpec((blk, d_in), lambda i: (i, 0)),
                  pl.BlockSpec((d_in, d_out), lambda i: (0, 0))],
        out_specs=pl.BlockSpec((blk, d_out), lambda i: (i, 0)),
        out_shape=jax.ShapeDtypeStruct((n, d_out), jnp.float32),
    )(x, W)


def _combine(partials, b):
    _, n, d = partials.shape
    blk = 1000
    b2 = b.reshape(1, d).astype(jnp.float32)

    def body(p_ref, b_ref, o_ref):
        o_ref[...] = p_ref[0] + p_ref[1] + b_ref[...]

    return pl.pallas_call(
        body,
        grid=(n // blk,),
        in_specs=[pl.BlockSpec((2, blk, d), lambda i: (0, i, 0)),
                  pl.BlockSpec((1, d), lambda i: (0, 0))],
        out_specs=pl.BlockSpec((blk, d), lambda i: (i, 0)),
        out_shape=jax.ShapeDtypeStruct((n, d), jnp.float32),
    )(partials, b2)


def _sc_aggregate(support, src3, dst3, w3):
    n, d = support.shape
    k_chunks = src3.shape[1]
    rows_per_tile = -(-(-(-n // _NS)) // 8) * 8
    n_pad = _NS * rows_per_tile
    mesh = plsc.VectorSubcoreMesh(core_axis_name="c", subcore_axis_name="s")

    @functools.partial(
        pl.kernel,
        out_type=jax.ShapeDtypeStruct((_NC, n_pad, d), jnp.float32),
        mesh=mesh,
        scratch_types=[
            pltpu.VMEM((k_chunks, _CHUNK), jnp.int32),
            pltpu.VMEM((k_chunks, _CHUNK), jnp.int32),
            pltpu.VMEM((k_chunks, _CHUNK), jnp.float32),
            pltpu.VMEM((_CHUNK, d), jnp.float32),
            pltpu.VMEM_SHARED((n_pad, d), jnp.float32),
            pltpu.SemaphoreType.DMA,
        ],
    )
    def agg(support_hbm, src_hbm, dst_hbm, w_hbm, out_hbm,
            src_v, dst_v, w_v, rows_v, acc_sh, sem):
        c = lax.axis_index("c")
        s = lax.axis_index("s")
        wid = s * _NC + c

        def zero_row(r, carry):
            for dd in range(d // _L):
                rows_v[r, pl.ds(dd * _L, _L)] = jnp.zeros((_L,), jnp.float32)
            return carry

        lax.fori_loop(0, _CHUNK, zero_row, 0)
        base = s * rows_per_tile
        nfull = rows_per_tile // _CHUNK
        for t in range(nfull):
            pltpu.sync_copy(rows_v, acc_sh.at[pl.ds(base + t * _CHUNK, _CHUNK)])
        rem = rows_per_tile - nfull * _CHUNK
        if rem:
            pltpu.sync_copy(rows_v.at[pl.ds(0, rem)],
                            acc_sh.at[pl.ds(base + nfull * _CHUNK, rem)])
        plsc.subcore_barrier()

        pltpu.sync_copy(src_hbm.at[wid], src_v)
        pltpu.sync_copy(dst_hbm.at[wid], dst_v)
        pltpu.sync_copy(w_hbm.at[wid], w_v)

        def chunk_body(j, carry):
            pltpu.async_copy(support_hbm.at[src_v.at[j]], rows_v, sem).wait()

            def scale_group(g, inner):
                wvec = w_v[j, pl.ds(g * _L, _L)]
                for rr in range(_L):
                    ws = wvec[rr]
                    r = g * _L + rr
                    for dd in range(d // _L):
                        sl = pl.ds(dd * _L, _L)
                        rows_v[r, sl] = rows_v[r, sl] * ws
                return inner

            lax.fori_loop(0, _CHUNK // _L, scale_group, 0)
            pltpu.sync_copy(rows_v, acc_sh.at[dst_v.at[j]], add=True)
            return carry

        lax.fori_loop(0, k_chunks, chunk_body, 0)
        plsc.subcore_barrier()

        pltpu.sync_copy(acc_sh.at[pl.ds(base, rows_per_tile)],
                        out_hbm.at[c, pl.ds(base, rows_per_tile)])

    return agg(support, src3, dst3, w3)[:, :n, :]


def kernel(input, edge_index, edge_weight, W, b):
    n = input.shape[0]
    e = edge_weight.shape[0]
    nw = _NC * _NS
    k_chunks = -(-e // (nw * _CHUNK))
    e_pad = nw * k_chunks * _CHUNK

    src = edge_index[0]
    dst = edge_index[1]
    pad = e_pad - e
    src3 = jnp.pad(src, (0, pad)).reshape(nw, k_chunks, _CHUNK)
    dst3 = jnp.pad(dst, (0, pad)).reshape(nw, k_chunks, _CHUNK)
    w3 = jnp.pad(edge_weight, (0, pad)).reshape(nw, k_chunks, _CHUNK)

    support = _matmul(input.astype(jnp.float32), W.astype(jnp.float32))
    partials = _sc_aggregate(support, src3, dst3, w3)
    return _combine(partials, b)
